# Initial kernel scaffold; baseline (speedup 1.0000x reference)
import jax
import jax.numpy as jnp
from jax.experimental import pallas as pl


def kernel(x, Wr, br, W1, b1, W2, b2):
    raise NotImplementedError("write your pallas kernel here")


# speedup vs baseline: 1.1300x; 1.1300x over previous
import functools

import jax
import jax.numpy as jnp
from jax.experimental import pallas as pl
import jax.experimental.pallas.tpu as pltpu

E = 8
TOPK = 2
NF = 8


def _moe_kernel(x_ref, wr_ref, br_ref, w1_ref, b1_ref, w2_ref, b2_ref,
                out_ref, combine_ref):
    e = pl.program_id(0)
    j = pl.program_id(1)

    @pl.when(jnp.logical_and(e == 0, j == 0))
    def _router():
        xf = x_ref[...]
        logits = jnp.dot(xf, wr_ref[...],
                         preferred_element_type=jnp.float32) + br_ref[...]
        m = jnp.max(logits, axis=-1, keepdims=True)
        ex = jnp.exp(logits - m)
        probs = ex / jnp.sum(ex, axis=-1, keepdims=True)
        col = jax.lax.broadcasted_iota(jnp.int32, probs.shape, 1)
        big = jnp.int32(E + 1)
        m1 = jnp.max(probs, axis=-1, keepdims=True)
        a1 = jnp.min(jnp.where(probs == m1, col, big), axis=-1, keepdims=True)
        p2 = jnp.where(col == a1, -jnp.inf, probs)
        m2 = jnp.max(p2, axis=-1, keepdims=True)
        a2 = jnp.min(jnp.where(p2 == m2, col, big), axis=-1, keepdims=True)
        denom = m1 + m2
        w1n = m1 / denom
        w2n = m2 / denom
        combine_ref[...] = jnp.where(col == a1, w1n,
                                     jnp.where(col == a2, w2n, 0.0))

    xb = x_ref[...].astype(jnp.bfloat16)
    w1c = w1_ref[0].astype(jnp.bfloat16)
    h = jnp.dot(xb, w1c, preferred_element_type=jnp.float32)
    h = jnp.maximum(h + b1_ref[0], 0.0)
    w2c = w2_ref[0].astype(jnp.bfloat16)
    y = jnp.dot(h.astype(jnp.bfloat16), w2c,
                preferred_element_type=jnp.float32)

    y = jnp.where(j == 0, y + b2_ref[0], y)

    cmb = combine_ref[...]
    ccol = jax.lax.broadcasted_iota(jnp.int32, cmb.shape, 1)
    c = jnp.sum(jnp.where(ccol == e, cmb, 0.0), axis=1, keepdims=True)
    contrib = y * c

    @pl.when(jnp.logical_and(e == 0, j == 0))
    def _init():
        out_ref[...] = contrib

    @pl.when(jnp.logical_not(jnp.logical_and(e == 0, j == 0)))
    def _acc():
        out_ref[...] = out_ref[...] + contrib


@functools.partial(jax.jit, static_argnames=())
def _moe(x2d, Wr, br2, W1, b1, W2, b2):
    T, D = x2d.shape
    F = W1.shape[2]
    FC = F // NF
    grid = (E, NF)
    out = pl.pallas_call(
        _moe_kernel,
        grid=grid,
        in_specs=[
            pl.BlockSpec((T, D), lambda e, j: (0, 0)),
            pl.BlockSpec((D, E), lambda e, j: (0, 0)),
            pl.BlockSpec((1, E), lambda e, j: (0, 0)),
            pl.BlockSpec((1, D, FC), lambda e, j: (e, 0, j)),
            pl.BlockSpec((1, 1, FC), lambda e, j: (e, 0, j)),
            pl.BlockSpec((1, FC, D), lambda e, j: (e, j, 0)),
            pl.BlockSpec((1, 1, D), lambda e, j: (e, 0, 0)),
        ],
        out_specs=pl.BlockSpec((T, D), lambda e, j: (0, 0)),
        out_shape=jax.ShapeDtypeStruct((T, D), jnp.float32),
        scratch_shapes=[pltpu.VMEM((T, E), jnp.float32)],
        compiler_params=pltpu.CompilerParams(
            dimension_semantics=("arbitrary", "arbitrary"),
        ),
    )(x2d, Wr, br2, W1, b1, W2, b2)
    return out


def kernel(x, Wr, br, W1, b1, W2, b2):
    B, S, D = x.shape
    x2d = x.reshape(B * S, D)
    out = _moe(x2d, Wr, br.reshape(1, E),
               W1, b1.reshape(E, 1, -1), W2, b2.reshape(E, 1, -1))
    return out.reshape(B, S, D)
